# baseline (speedup 1.0000x reference)
import functools

import jax
import jax.numpy as jnp
from jax import lax
from jax.experimental import pallas as pl
from jax.experimental.pallas import tpu as pltpu
from jax.experimental.pallas import tpu_sc as plsc

B = 128
N = 100000

RA = 16
JA = B // RA
NSLOT = 3

NW = 32
CH = 20000
RPW = B // NW


def _tc_argmax_body(x_hbm, idx_hbm, rbuf, ibuf, rsem, isem):
    def _read(j, slot):
        pltpu.make_async_copy(
            x_hbm.at[pl.ds(j * RA, RA), :], rbuf.at[slot], rsem.at[slot]
        ).start()

    def _rwait(j, slot):
        pltpu.make_async_copy(
            x_hbm.at[pl.ds(j * RA, RA), :], rbuf.at[slot], rsem.at[slot]
        ).wait()

    _read(0, 0)
    _read(1, 1)

    def _step(j, _):
        slot = lax.rem(j, NSLOT)
        _rwait(j, slot)

        @pl.when(j + 2 < JA)
        def _():
            _read(j + 2, lax.rem(j + 2, NSLOT))

        x = rbuf[slot]
        col = lax.broadcasted_iota(jnp.int32, (RA, N), 1)
        bmax = jnp.max(x, axis=1, keepdims=True)
        bidx = jnp.min(jnp.where(x == bmax, col, N), axis=1, keepdims=True)
        row = lax.broadcasted_iota(jnp.int32, (RA, 1), 0) + j * RA
        ibuf[pl.ds(j * RA, RA), :] = bidx + row * N
        return 0

    lax.fori_loop(0, JA, _step, 0)

    pltpu.make_async_copy(ibuf, idx_hbm, isem).start()
    pltpu.make_async_copy(ibuf, idx_hbm, isem).wait()


_tc_argmax = pl.pallas_call(
    _tc_argmax_body,
    in_specs=[pl.BlockSpec(memory_space=pl.ANY)],
    out_specs=pl.BlockSpec(memory_space=pl.ANY),
    out_shape=jax.ShapeDtypeStruct((B, 1), jnp.int32),
    scratch_shapes=[
        pltpu.VMEM((NSLOT, RA, N), jnp.float32),
        pltpu.VMEM((B, 1), jnp.int32),
        pltpu.SemaphoreType.DMA((NSLOT,)),
        pltpu.SemaphoreType.DMA,
    ],
)


@functools.cache
def _make_sc_zeros():
    mesh = plsc.VectorSubcoreMesh(core_axis_name="c", subcore_axis_name="s")

    @functools.partial(
        pl.kernel,
        out_type=jax.ShapeDtypeStruct((B * N,), jnp.float32),
        mesh=mesh,
        scratch_types=[
            pltpu.VMEM((CH,), jnp.float32),
            pltpu.SemaphoreType.DMA,
        ],
    )
    def _sc_zeros(out_hbm, zbuf, sem):
        c = lax.axis_index("c")
        s = lax.axis_index("s")
        wid = s * 2 + c

        @pl.loop(0, CH // 16, unroll=8)
        def _(i):
            zbuf[pl.ds(i * 16, 16)] = jnp.zeros((16,), jnp.float32)

        base = wid * RPW * N
        copies = [
            pltpu.async_copy(
                zbuf, out_hbm.at[pl.ds(base + k * CH, CH)], sem)
            for k in range(RPW * N // CH)
        ]
        for cp in copies:
            cp.wait()

    return _sc_zeros


@functools.cache
def _make_sc_ones():
    mesh = plsc.VectorSubcoreMesh(core_axis_name="c", subcore_axis_name="s")

    @functools.partial(
        pl.kernel,
        mesh=mesh,
        scratch_types=[
            pltpu.VMEM((B,), jnp.int32),
            pltpu.VMEM((B,), jnp.float32),
            pltpu.SemaphoreType.DMA,
        ],
    )
    def _sc_ones(out_hbm, idx_hbm, idx_v, ones_v, sem):
        c = lax.axis_index("c")
        s = lax.axis_index("s")

        @pl.when((c == 0) & (s == 0))
        def _():
            pltpu.sync_copy(idx_hbm, idx_v)
            for i in range(B // 16):
                ones_v[pl.ds(i * 16, 16)] = jnp.full((16,), 1.0, jnp.float32)
            pltpu.async_copy(ones_v, out_hbm.at[idx_v], sem).wait()

    return _sc_ones


def kernel(logits, use_gpu):
    del use_gpu
    idx = _tc_argmax(logits)
    flat = _make_sc_zeros()()
    flat_ref = jax.new_ref(flat)
    _make_sc_ones()(flat_ref, idx.reshape(B))
    return jax.freeze(flat_ref).reshape(B, N)
